# baseline (speedup 1.0000x reference)
import functools

import jax
import jax.numpy as jnp
from jax import lax
from jax.experimental import pallas as pl
from jax.experimental.pallas import tpu as pltpu
from jax.experimental.pallas import tpu_sc as plsc

NUM_CORES = 2
NUM_SUBCORES = 16
NUM_WORKERS = NUM_CORES * NUM_SUBCORES
LANES = 16
CHUNK = 128


@functools.cache
def _build(batch, k_dim):
    assert batch % (NUM_WORKERS * CHUNK) == 0
    b_per_w = batch // NUM_WORKERS
    n_chunks = b_per_w // CHUNK
    mesh = plsc.VectorSubcoreMesh(core_axis_name="c", subcore_axis_name="s")

    @functools.partial(
        pl.kernel,
        out_type=jax.ShapeDtypeStruct((batch,), jnp.float32),
        mesh=mesh,
        scratch_types=[
            pltpu.VMEM((CHUNK, 2), jnp.int32),
            pltpu.VMEM((CHUNK,), jnp.int32),
            pltpu.VMEM((CHUNK,), jnp.int32),
            pltpu.VMEM((k_dim, CHUNK), jnp.int32),
            pltpu.VMEM((k_dim, CHUNK), jnp.int32),
            pltpu.VMEM((k_dim, CHUNK), jnp.float32),
            pltpu.VMEM((k_dim, CHUNK), jnp.float32),
            pltpu.VMEM((CHUNK,), jnp.float32),
            pltpu.VMEM((CHUNK,), jnp.float32),
            pltpu.VMEM((CHUNK,), jnp.float32),
            pltpu.VMEM((LANES,), jnp.float32),
            pltpu.SemaphoreType.DMA,
        ],
        compiler_params=pltpu.CompilerParams(
            needs_layout_passes=False, use_tc_tiling_on_sc=False),
    )
    def fm(idx_h, w0_h, w_h, v1_h, out_h,
           idx_v, u_v, m_v, iu_v, im_v, vut_v, vmt_v, wu_v, wm_v, out_v,
           w0_v, sem):
        wid = lax.axis_index("s") * NUM_CORES + lax.axis_index("c")
        base = wid * b_per_w
        zeros = jnp.zeros((LANES,), jnp.int32)
        iota = lax.iota(jnp.int32, LANES)
        rt_zeros = jnp.minimum(iota, 0)

        pltpu.sync_copy(w0_h, w0_v.at[pl.ds(0, 1)])
        w0_vec = w0_v[...][rt_zeros]

        def chunk_body(c, carry):
            off = base + c * CHUNK
            pltpu.sync_copy(idx_h.at[pl.ds(off, CHUNK)], idx_v)
            for g in range(CHUNK // LANES):
                rows = iota + g * LANES
                u_v[pl.ds(g * LANES, LANES)] = plsc.load_gather(
                    idx_v, [rows, zeros])
                m_v[pl.ds(g * LANES, LANES)] = plsc.load_gather(
                    idx_v, [rows, zeros + 1])
            for g in range(CHUNK // LANES):
                b0 = g * LANES
                u_k = u_v[pl.ds(b0, LANES)] * k_dim
                m_k = m_v[pl.ds(b0, LANES)] * k_dim
                for k in range(k_dim):
                    iu_v[k, pl.ds(b0, LANES)] = u_k + k
                    im_v[k, pl.ds(b0, LANES)] = m_k + k
            cps = [
                pltpu.async_copy(w_h.at[u_v], wu_v, sem),
                pltpu.async_copy(w_h.at[m_v], wm_v, sem),
            ]
            for k in range(k_dim):
                cps.append(pltpu.async_copy(v1_h.at[iu_v.at[k]],
                                            vut_v.at[k], sem))
                cps.append(pltpu.async_copy(v1_h.at[im_v.at[k]],
                                            vmt_v.at[k], sem))
            for cp in cps:
                cp.wait()
            for g in range(CHUNK // LANES):
                b0 = g * LANES
                acc = wu_v[pl.ds(b0, LANES)] + wm_v[pl.ds(b0, LANES)] + w0_vec
                for k in range(k_dim):
                    acc = acc + (vut_v[k, pl.ds(b0, LANES)]
                                 * vmt_v[k, pl.ds(b0, LANES)])
                out_v[pl.ds(b0, LANES)] = acc
            pltpu.sync_copy(out_v, out_h.at[pl.ds(off, CHUNK)])
            return carry

        lax.fori_loop(0, n_chunks, chunk_body, 0)

    return fm


def kernel(idx, w0, w, V):
    return _build(idx.shape[0], V.shape[1])(idx, w0, w, V.reshape(-1))


# speedup vs baseline: 1.0294x; 1.0294x over previous
import functools

import jax
import jax.numpy as jnp
from jax import lax
from jax.experimental import pallas as pl
from jax.experimental.pallas import tpu as pltpu
from jax.experimental.pallas import tpu_sc as plsc

NUM_CORES = 2
NUM_SUBCORES = 16
NUM_WORKERS = NUM_CORES * NUM_SUBCORES
LANES = 16
CHUNK = 128


@functools.cache
def _build(batch, k_dim):
    assert batch % (NUM_WORKERS * CHUNK) == 0
    assert (k_dim % 16) + k_dim <= 32
    b_per_w = batch // NUM_WORKERS
    n_chunks = b_per_w // CHUNK
    mesh = plsc.VectorSubcoreMesh(core_axis_name="c", subcore_axis_name="s")

    @functools.partial(
        pl.kernel,
        out_type=jax.ShapeDtypeStruct((batch,), jnp.float32),
        mesh=mesh,
        scratch_types=[
            pltpu.VMEM((CHUNK, 2), jnp.int32),
            pltpu.VMEM((CHUNK,), jnp.int32),
            pltpu.VMEM((CHUNK,), jnp.int32),
            pltpu.VMEM((CHUNK,), jnp.int32),
            pltpu.VMEM((CHUNK,), jnp.int32),
            pltpu.VMEM((CHUNK,), jnp.int32),
            pltpu.VMEM((CHUNK,), jnp.int32),
            pltpu.VMEM((2 * CHUNK, LANES), jnp.float32),
            pltpu.VMEM((2 * CHUNK, LANES), jnp.float32),
            pltpu.VMEM((CHUNK,), jnp.float32),
            pltpu.VMEM((CHUNK,), jnp.float32),
            pltpu.VMEM((CHUNK,), jnp.float32),
            pltpu.VMEM((LANES,), jnp.float32),
            pltpu.SemaphoreType.DMA,
        ],
        compiler_params=pltpu.CompilerParams(
            needs_layout_passes=False, use_tc_tiling_on_sc=False),
    )
    def fm(idx_h, w0_h, w_h, v16_h, out_h,
           idx_v, u_v, m_v, ua_v, ub_v, ma_v, mb_v, du_v, dm_v,
           wu_v, wm_v, out_v, w0_v, sem):
        wid = lax.axis_index("s") * NUM_CORES + lax.axis_index("c")
        base = wid * b_per_w
        zeros = jnp.zeros((LANES,), jnp.int32)
        iota = lax.iota(jnp.int32, LANES)
        rt_zeros = jnp.minimum(iota, 0)

        pltpu.sync_copy(w0_h, w0_v.at[pl.ds(0, 1)])
        w0_vec = w0_v[...][rt_zeros]

        def chunk_body(c, carry):
            off = base + c * CHUNK
            pltpu.sync_copy(idx_h.at[pl.ds(off, CHUNK)], idx_v)
            for g in range(CHUNK // LANES):
                b0 = g * LANES
                rows = iota + b0
                uu = plsc.load_gather(idx_v, [rows, zeros])
                mm = plsc.load_gather(idx_v, [rows, zeros + 1])
                u_v[pl.ds(b0, LANES)] = uu
                m_v[pl.ds(b0, LANES)] = mm
                ua = (uu * k_dim) >> 4
                ma = (mm * k_dim) >> 4
                ua_v[pl.ds(b0, LANES)] = ua
                ub_v[pl.ds(b0, LANES)] = ua + 1
                ma_v[pl.ds(b0, LANES)] = ma
                mb_v[pl.ds(b0, LANES)] = ma + 1
            cps = [
                pltpu.async_copy(w_h.at[u_v], wu_v, sem),
                pltpu.async_copy(w_h.at[m_v], wm_v, sem),
                pltpu.async_copy(v16_h.at[ua_v], du_v.at[pl.ds(0, CHUNK)], sem),
                pltpu.async_copy(v16_h.at[ub_v], du_v.at[pl.ds(CHUNK, CHUNK)], sem),
                pltpu.async_copy(v16_h.at[ma_v], dm_v.at[pl.ds(0, CHUNK)], sem),
                pltpu.async_copy(v16_h.at[mb_v], dm_v.at[pl.ds(CHUNK, CHUNK)], sem),
            ]
            for cp in cps:
                cp.wait()
            for g in range(CHUNK // LANES):
                b0 = g * LANES
                rows = iota + b0
                s_u = (u_v[pl.ds(b0, LANES)] * k_dim) & 15
                s_m = (m_v[pl.ds(b0, LANES)] * k_dim) & 15
                acc = wu_v[pl.ds(b0, LANES)] + wm_v[pl.ds(b0, LANES)] + w0_vec
                for k in range(k_dim):
                    wu_i = s_u + k
                    wm_i = s_m + k
                    a = plsc.load_gather(
                        du_v, [rows + ((wu_i >> 4) << 7), wu_i & 15])
                    b = plsc.load_gather(
                        dm_v, [rows + ((wm_i >> 4) << 7), wm_i & 15])
                    acc = acc + a * b
                out_v[pl.ds(b0, LANES)] = acc
            pltpu.sync_copy(out_v, out_h.at[pl.ds(off, CHUNK)])
            return carry

        lax.fori_loop(0, n_chunks, chunk_body, 0)

    return fm


def kernel(idx, w0, w, V):
    return _build(idx.shape[0], V.shape[1])(
        idx, w0, w, V.reshape(-1, 16))


# speedup vs baseline: 1.0324x; 1.0028x over previous
import functools

import jax
import jax.numpy as jnp
from jax import lax
from jax.experimental import pallas as pl
from jax.experimental.pallas import tpu as pltpu
from jax.experimental.pallas import tpu_sc as plsc

NUM_CORES = 2
NUM_SUBCORES = 16
NUM_WORKERS = NUM_CORES * NUM_SUBCORES
LANES = 16
CHUNK = 128


@functools.cache
def _build(batch, k_dim):
    assert batch % (NUM_WORKERS * CHUNK) == 0
    assert (k_dim % 16) + k_dim <= 32
    b_per_w = batch // NUM_WORKERS
    n_chunks = b_per_w // CHUNK
    mesh = plsc.VectorSubcoreMesh(core_axis_name="c", subcore_axis_name="s")

    @functools.partial(
        pl.kernel,
        out_type=jax.ShapeDtypeStruct((batch,), jnp.float32),
        mesh=mesh,
        scratch_types=[
            pltpu.VMEM((CHUNK, 2), jnp.int32),
            pltpu.VMEM((CHUNK,), jnp.int32),
            pltpu.VMEM((CHUNK,), jnp.int32),
            pltpu.VMEM((CHUNK,), jnp.int32),
            pltpu.VMEM((CHUNK,), jnp.int32),
            pltpu.VMEM((CHUNK,), jnp.int32),
            pltpu.VMEM((CHUNK,), jnp.int32),
            pltpu.VMEM((2 * CHUNK, LANES), jnp.float32),
            pltpu.VMEM((2 * CHUNK, LANES), jnp.float32),
            pltpu.VMEM((CHUNK,), jnp.float32),
            pltpu.VMEM((CHUNK,), jnp.float32),
            pltpu.VMEM((CHUNK,), jnp.float32),
            pltpu.VMEM((LANES,), jnp.float32),
            pltpu.SemaphoreType.DMA,
        ],
        compiler_params=pltpu.CompilerParams(
            needs_layout_passes=False, use_tc_tiling_on_sc=False),
    )
    def fm(idx_h, w0_h, w_h, v16_h, out_h,
           idx_v, u_v, m_v, ua_v, ub_v, ma_v, mb_v, du_v, dm_v,
           wu_v, wm_v, out_v, w0_v, sem):
        wid = lax.axis_index("s") * NUM_CORES + lax.axis_index("c")
        base = wid * b_per_w
        zeros = jnp.zeros((LANES,), jnp.int32)
        iota = lax.iota(jnp.int32, LANES)
        rt_zeros = jnp.minimum(iota, 0)

        pltpu.sync_copy(w0_h, w0_v.at[pl.ds(0, 1)])
        w0_vec = w0_v[...][rt_zeros]

        def chunk_body(c, carry):
            off = base + c * CHUNK
            pltpu.sync_copy(idx_h.at[pl.ds(off, CHUNK)], idx_v)

            def pre_g(g, carry2):
                b0 = g * LANES
                rows = iota + b0
                uu = plsc.load_gather(idx_v, [rows, zeros])
                mm = plsc.load_gather(idx_v, [rows, zeros + 1])
                u_v[pl.ds(b0, LANES)] = uu
                m_v[pl.ds(b0, LANES)] = mm
                ua = (uu * k_dim) >> 4
                ma = (mm * k_dim) >> 4
                ua_v[pl.ds(b0, LANES)] = ua
                ub_v[pl.ds(b0, LANES)] = ua + 1
                ma_v[pl.ds(b0, LANES)] = ma
                mb_v[pl.ds(b0, LANES)] = ma + 1
                return carry2

            lax.fori_loop(0, CHUNK // LANES, pre_g, 0)
            cps = [
                pltpu.async_copy(w_h.at[u_v], wu_v, sem),
                pltpu.async_copy(w_h.at[m_v], wm_v, sem),
                pltpu.async_copy(v16_h.at[ua_v], du_v.at[pl.ds(0, CHUNK)], sem),
                pltpu.async_copy(v16_h.at[ub_v], du_v.at[pl.ds(CHUNK, CHUNK)], sem),
                pltpu.async_copy(v16_h.at[ma_v], dm_v.at[pl.ds(0, CHUNK)], sem),
                pltpu.async_copy(v16_h.at[mb_v], dm_v.at[pl.ds(CHUNK, CHUNK)], sem),
            ]
            for cp in cps:
                cp.wait()

            def comp_g(g, carry2):
                b0 = g * LANES
                rows = iota + b0
                s_u = (u_v[pl.ds(b0, LANES)] * k_dim) & 15
                s_m = (m_v[pl.ds(b0, LANES)] * k_dim) & 15
                acc0 = wu_v[pl.ds(b0, LANES)] + wm_v[pl.ds(b0, LANES)] + w0_vec

                def comp_k(k, acc):
                    wu_i = s_u + k
                    wm_i = s_m + k
                    a = plsc.load_gather(
                        du_v, [rows + ((wu_i >> 4) << 7), wu_i & 15])
                    b = plsc.load_gather(
                        dm_v, [rows + ((wm_i >> 4) << 7), wm_i & 15])
                    return acc + a * b

                out_v[pl.ds(b0, LANES)] = lax.fori_loop(
                    0, k_dim, comp_k, acc0)
                return carry2

            lax.fori_loop(0, CHUNK // LANES, comp_g, 0)
            pltpu.sync_copy(out_v, out_h.at[pl.ds(off, CHUNK)])
            return carry

        lax.fori_loop(0, n_chunks, chunk_body, 0)

    return fm


def kernel(idx, w0, w, V):
    return _build(idx.shape[0], V.shape[1])(
        idx, w0, w, V.reshape(-1, 16))


# speedup vs baseline: 2.6607x; 2.5773x over previous
import functools

import jax
import jax.numpy as jnp
from jax import lax
from jax.experimental import pallas as pl
from jax.experimental.pallas import tpu as pltpu
from jax.experimental.pallas import tpu_sc as plsc

NUM_CORES = 2
NUM_SUBCORES = 16
NUM_WORKERS = NUM_CORES * NUM_SUBCORES
LANES = 16
CHUNK = 128


@functools.cache
def _build(batch, k_dim):
    assert batch % (NUM_WORKERS * CHUNK) == 0
    b_per_w = batch // NUM_WORKERS
    n_chunks = b_per_w // CHUNK
    mesh = plsc.VectorSubcoreMesh(core_axis_name="c", subcore_axis_name="s")

    @functools.partial(
        pl.kernel,
        out_type=jax.ShapeDtypeStruct((batch,), jnp.float32),
        mesh=mesh,
        scratch_types=[
            pltpu.VMEM((2 * CHUNK,), jnp.int32),
            pltpu.VMEM((CHUNK,), jnp.int32),
            pltpu.VMEM((CHUNK,), jnp.int32),
            pltpu.VMEM((CHUNK, k_dim), jnp.float32),
            pltpu.VMEM((CHUNK, k_dim), jnp.float32),
            pltpu.VMEM((CHUNK,), jnp.float32),
            pltpu.VMEM((CHUNK,), jnp.float32),
            pltpu.VMEM((CHUNK,), jnp.float32),
            pltpu.VMEM((LANES,), jnp.float32),
            pltpu.SemaphoreType.DMA,
            pltpu.SemaphoreType.DMA,
        ],
        compiler_params=pltpu.CompilerParams(
            needs_layout_passes=False, use_tc_tiling_on_sc=True),
    )
    def fm(idxf_h, w0_h, w_h, v_h, out_h,
           idx_v, u_v, m_v, du_v, dm_v, wu_v, wm_v, out_v, w0_v, sem, sem2):
        wid = lax.axis_index("s") * NUM_CORES + lax.axis_index("c")
        base = wid * b_per_w
        zeros = jnp.zeros((LANES,), jnp.int32)
        iota = lax.iota(jnp.int32, LANES)
        rt_zeros = jnp.minimum(iota, 0)

        pltpu.sync_copy(w0_h, w0_v.at[pl.ds(0, 1)])
        w0_vec = w0_v[...][rt_zeros]

        def chunk_body(c, carry):
            off = base + c * CHUNK
            pltpu.sync_copy(idxf_h.at[pl.ds(2 * off, 2 * CHUNK)], idx_v)

            def pre_g(g, carry2):
                b0 = g * LANES
                rows = (iota + b0) * 2
                u_v[pl.ds(b0, LANES)] = plsc.load_gather(idx_v, [rows])
                m_v[pl.ds(b0, LANES)] = plsc.load_gather(idx_v, [rows + 1])
                return carry2

            lax.fori_loop(0, CHUNK // LANES, pre_g, 0)
            cpw = [
                pltpu.async_copy(w_h.at[u_v], wu_v, sem2),
                pltpu.async_copy(w_h.at[m_v], wm_v, sem2),
            ]

            def fire(g, carry2):
                b0 = g * LANES
                u16 = u_v[pl.ds(b0, LANES)]
                m16 = m_v[pl.ds(b0, LANES)]
                for lane in range(LANES):
                    pltpu.make_async_copy(
                        v_h.at[pl.ds(u16[lane], 1), pl.ds(0, k_dim)],
                        du_v.at[pl.ds(b0 + lane, 1), pl.ds(0, k_dim)],
                        sem).start()
                    pltpu.make_async_copy(
                        v_h.at[pl.ds(m16[lane], 1), pl.ds(0, k_dim)],
                        dm_v.at[pl.ds(b0 + lane, 1), pl.ds(0, k_dim)],
                        sem).start()
                return carry2

            lax.fori_loop(0, CHUNK // LANES, fire, 0)
            pltpu.make_async_copy(v_h.at[pl.ds(0, CHUNK), pl.ds(0, k_dim)],
                                  du_v, sem).wait()
            pltpu.make_async_copy(v_h.at[pl.ds(0, CHUNK), pl.ds(0, k_dim)],
                                  dm_v, sem).wait()
            for cp in cpw:
                cp.wait()

            def comp_g(g, carry2):
                b0 = g * LANES
                rows = iota + b0
                acc0 = wu_v[pl.ds(b0, LANES)] + wm_v[pl.ds(b0, LANES)] + w0_vec

                def comp_k(k, a):
                    return (a + plsc.load_gather(du_v, [rows, zeros + k])
                            * plsc.load_gather(dm_v, [rows, zeros + k]))

                out_v[pl.ds(b0, LANES)] = lax.fori_loop(0, k_dim, comp_k, acc0)
                return carry2

            lax.fori_loop(0, CHUNK // LANES, comp_g, 0)
            pltpu.sync_copy(out_v, out_h.at[pl.ds(off, CHUNK)])
            return carry

        lax.fori_loop(0, n_chunks, chunk_body, 0)

    return fm


def kernel(idx, w0, w, V):
    return _build(idx.shape[0], V.shape[1])(idx.reshape(-1), w0, w, V)


# speedup vs baseline: 2.7604x; 1.0375x over previous
import functools

import jax
import jax.numpy as jnp
from jax import lax
from jax.experimental import pallas as pl
from jax.experimental.pallas import tpu as pltpu
from jax.experimental.pallas import tpu_sc as plsc

NUM_CORES = 2
NUM_SUBCORES = 16
NUM_WORKERS = NUM_CORES * NUM_SUBCORES
LANES = 16
CHUNK = 128


@functools.cache
def _build(batch, k_dim):
    assert batch % (NUM_WORKERS * CHUNK) == 0
    b_per_w = batch // NUM_WORKERS
    n_chunks = b_per_w // CHUNK
    mesh = plsc.VectorSubcoreMesh(core_axis_name="c", subcore_axis_name="s")

    @functools.partial(
        pl.kernel,
        out_type=jax.ShapeDtypeStruct((batch,), jnp.float32),
        mesh=mesh,
        scratch_types=[
            pltpu.VMEM((2, CHUNK), jnp.int32),
            pltpu.VMEM((CHUNK,), jnp.int32),
            pltpu.VMEM((CHUNK,), jnp.int32),
            pltpu.VMEM((CHUNK, k_dim), jnp.float32),
            pltpu.VMEM((CHUNK, k_dim), jnp.float32),
            pltpu.VMEM((CHUNK,), jnp.float32),
            pltpu.VMEM((CHUNK,), jnp.float32),
            pltpu.VMEM((CHUNK,), jnp.float32),
            pltpu.VMEM((LANES,), jnp.float32),
            pltpu.SemaphoreType.DMA,
            pltpu.SemaphoreType.DMA,
        ],
        compiler_params=pltpu.CompilerParams(
            needs_layout_passes=False, use_tc_tiling_on_sc=True),
    )
    def fm(idxt_h, w0_h, w_h, v_h, out_h,
           idx_v, u_v, m_v, du_v, dm_v, wu_v, wm_v, out_v, w0_v, sem, sem2):
        wid = lax.axis_index("s") * NUM_CORES + lax.axis_index("c")
        base = wid * b_per_w
        zeros = jnp.zeros((LANES,), jnp.int32)
        iota = lax.iota(jnp.int32, LANES)
        rt_zeros = jnp.minimum(iota, 0)

        pltpu.sync_copy(w0_h, w0_v.at[pl.ds(0, 1)])
        w0_vec = w0_v[...][rt_zeros]

        def chunk_body(c, carry):
            off = base + c * CHUNK
            pltpu.sync_copy(idxt_h.at[pl.ds(0, 2), pl.ds(off, CHUNK)], idx_v)

            def pre_g(g, carry2):
                b0 = g * LANES
                u_v[pl.ds(b0, LANES)] = idx_v[0, pl.ds(b0, LANES)]
                m_v[pl.ds(b0, LANES)] = idx_v[1, pl.ds(b0, LANES)]
                return carry2

            lax.fori_loop(0, CHUNK // LANES, pre_g, 0)
            cpw = [
                pltpu.async_copy(w_h.at[u_v], wu_v, sem2),
                pltpu.async_copy(w_h.at[m_v], wm_v, sem2),
            ]

            def fire(g, carry2):
                b0 = g * LANES
                u16 = u_v[pl.ds(b0, LANES)]
                m16 = m_v[pl.ds(b0, LANES)]
                for lane in range(LANES):
                    pltpu.make_async_copy(
                        v_h.at[pl.ds(u16[lane], 1), pl.ds(0, k_dim)],
                        du_v.at[pl.ds(b0 + lane, 1), pl.ds(0, k_dim)],
                        sem).start()
                    pltpu.make_async_copy(
                        v_h.at[pl.ds(m16[lane], 1), pl.ds(0, k_dim)],
                        dm_v.at[pl.ds(b0 + lane, 1), pl.ds(0, k_dim)],
                        sem).start()
                return carry2

            lax.fori_loop(0, CHUNK // LANES, fire, 0)
            pltpu.make_async_copy(v_h.at[pl.ds(0, CHUNK), pl.ds(0, k_dim)],
                                  du_v, sem).wait()
            pltpu.make_async_copy(v_h.at[pl.ds(0, CHUNK), pl.ds(0, k_dim)],
                                  dm_v, sem).wait()
            for cp in cpw:
                cp.wait()

            def comp_g(g, carry2):
                b0 = g * LANES
                rows = iota + b0
                acc0 = wu_v[pl.ds(b0, LANES)] + wm_v[pl.ds(b0, LANES)] + w0_vec

                def comp_k(k, a):
                    return (a + plsc.load_gather(du_v, [rows, zeros + k])
                            * plsc.load_gather(dm_v, [rows, zeros + k]))

                out_v[pl.ds(b0, LANES)] = lax.fori_loop(0, k_dim, comp_k, acc0)
                return carry2

            lax.fori_loop(0, CHUNK // LANES, comp_g, 0)
            pltpu.sync_copy(out_v, out_h.at[pl.ds(off, CHUNK)])
            return carry

        lax.fori_loop(0, n_chunks, chunk_body, 0)

    return fm


def kernel(idx, w0, w, V):
    return _build(idx.shape[0], V.shape[1])(idx.T, w0, w, V)
